# baseline (speedup 1.0000x reference)
import functools

import jax
import jax.numpy as jnp
from jax import lax
from jax.experimental import pallas as pl


def _mm_half0(u_ref, it_ref, o_ref):
  o_ref[...] = lax.dot_general(
      u_ref[...], it_ref[...],
      dimension_numbers=(((1,), (1,)), ((), ())),
      preferred_element_type=jnp.float32,
  )


def _mm_half1(s_ref, u_ref, it_ref, o_ref):
  del s_ref
  o_ref[...] = lax.dot_general(
      u_ref[...], it_ref[...],
      dimension_numbers=(((1,), (1,)), ((), ())),
      preferred_element_type=jnp.float32,
  )


def _tc_scores(emb, batch, dim):
  half = batch // 2

  s0 = pl.pallas_call(
      _mm_half0,
      grid=(1,),
      in_specs=[
          pl.BlockSpec((half, dim), lambda i: (0, 0)),
          pl.BlockSpec((batch, dim), lambda i: (1, 0)),
      ],
      out_specs=pl.BlockSpec((half, batch), lambda i: (0, 0)),
      out_shape=jax.ShapeDtypeStruct((batch, batch), jnp.float32),
  )(emb, emb)

  s1 = pl.pallas_call(
      _mm_half1,
      grid=(1,),
      in_specs=[
          pl.BlockSpec(memory_space=pl.ANY),
          pl.BlockSpec((half, dim), lambda i: (1, 0)),
          pl.BlockSpec((batch, dim), lambda i: (1, 0)),
      ],
      out_specs=pl.BlockSpec((half, batch), lambda i: (1, 0)),
      out_shape=jax.ShapeDtypeStruct((batch, batch), jnp.float32),
      input_output_aliases={0: 0},
  )(s0, emb, emb)
  return s1


@jax.jit
def kernel(id_embedding, user_tensor, item_tensor):
  batch = user_tensor.shape[0]
  dim = id_embedding.shape[1]
  idx = jnp.concatenate(
      [user_tensor.astype(jnp.int32), item_tensor.astype(jnp.int32)])
  emb = jnp.take(id_embedding, idx, axis=0)
  return _tc_scores(emb, batch, dim)
